# Initial kernel scaffold; baseline (speedup 1.0000x reference)
import jax
import jax.numpy as jnp
from jax.experimental import pallas as pl


def kernel(x, row, col, data, indices, W, b):
    raise NotImplementedError("write your pallas kernel here")


# speedup vs baseline: 3.7370x; 3.7370x over previous
import functools

import jax
import jax.numpy as jnp
from jax import lax
from jax.experimental import pallas as pl
from jax.experimental.pallas import tpu as pltpu
from jax.experimental.pallas import tpu_sc as plsc

B, N_IN, N_OUT, C, S = 4, 25000, 50000, 64, 9
NNZ = 100000
NC, NS, L = 2, 16, 16

HALF = N_OUT // NC
APAD = 25088
AROWS_T = APAD // NS
NNZ_PAD = 102400
E_T = NNZ_PAD // NS
ECH = 256
S10 = 10
VPAD = 51200
RP10 = VPAD * S10
R_T = RP10 // (NC * NS)
RCH = 640
GROWS = B * VPAD * S10 // 2
VB = 400
KS = S10 // 2


def _pool_body(x_hbm, colp, rowp, datap, pooled_hbm,
               acc, xrows, colbuf, rowbuf, databuf, lrbuf, sem):
    c = lax.axis_index("c")
    t = lax.axis_index("s")
    base_local = HALF * c

    zv = jnp.zeros((L,), jnp.float32)
    for b in range(B):
        for r in range(ECH):
            for j in range(C // L):
                xrows[r, pl.ds(j * L, L)] = zv
        for z6 in range(AROWS_T // ECH):
            pltpu.sync_copy(xrows, acc.at[pl.ds(t * AROWS_T + z6 * ECH, ECH)])
        pltpu.sync_copy(
            xrows.at[pl.ds(0, AROWS_T % ECH)],
            acc.at[pl.ds(t * AROWS_T + (AROWS_T // ECH) * ECH, AROWS_T % ECH)])
        plsc.subcore_barrier()

        @pl.loop(0, E_T // ECH)
        def _chunk(k):
            ebase = t * E_T + k * ECH
            pltpu.sync_copy(colp.at[pl.ds(ebase, ECH)], colbuf)
            pltpu.sync_copy(rowp.at[pl.ds(ebase, ECH)], rowbuf)
            pltpu.sync_copy(datap.at[pl.ds(ebase, ECH)], databuf)
            pltpu.async_copy(x_hbm.at[b].at[colbuf], xrows, sem).wait()
            for i in range(ECH // L):
                rv = rowbuf[pl.ds(i * L, L)]
                lr = rv - base_local
                ok = (lr >= 0) & (lr < HALF)
                lrbuf[pl.ds(i * L, L)] = jnp.where(ok, lr, HALF)
            @pl.loop(0, ECH)
            def _edge(e):
                dvec = plsc.load_gather(databuf, [jnp.full((L,), e, jnp.int32)])
                for j in range(C // L):
                    xrows[e, pl.ds(j * L, L)] = xrows[e, pl.ds(j * L, L)] * dvec
            pltpu.sync_copy(xrows, acc.at[lrbuf], add=True)

        plsc.subcore_barrier()
        pltpu.sync_copy(acc.at[pl.ds(t * AROWS_T, AROWS_T)],
                        pooled_hbm.at[b].at[c].at[pl.ds(t * AROWS_T, AROWS_T)])


def _spiral_body(pooled_hbm, idx_hbm, g_hbm, idxbuf, grows, sem):
    c = lax.axis_index("c")
    t = lax.axis_index("s")
    wid = t * NC + c
    rbase = wid * R_T
    gview = g_hbm

    @pl.loop(0, R_T // RCH)
    def _chunk(k):
        base = rbase + k * RCH
        pltpu.sync_copy(idx_hbm.at[pl.ds(base, RCH)], idxbuf)
        for i in range(RCH // L):
            iv = idxbuf[pl.ds(i * L, L)]
            idxbuf[pl.ds(i * L, L)] = jnp.where(iv >= HALF, iv + (APAD - HALF), iv)
        for b in range(B):
            pltpu.async_copy(pooled_hbm.at[b].at[idxbuf], grows, sem).wait()
            pltpu.sync_copy(grows, gview.at[b].at[pl.ds(base, RCH)])


def _dense_body(g_ref, w_ref, bias_ref, out_ref):
    z = jax.lax.dot_general(g_ref[0, 0], w_ref[0], (((1,), (0,)), ((), ())),
                            preferred_element_type=jnp.float32)
    for ks in range(1, KS):
        z = z + jax.lax.dot_general(g_ref[0, ks], w_ref[ks],
                                    (((1,), (0,)), ((), ())),
                                    preferred_element_type=jnp.float32)
    z = z + bias_ref[...]
    out_ref[0] = jnp.where(z > 0, z, jnp.exp(z) - 1.0)


@functools.partial(jax.jit, static_argnums=())
def kernel(x, row, col, data, indices, W, b):
    mesh = plsc.VectorSubcoreMesh(core_axis_name="c", subcore_axis_name="s",
                                  num_cores=NC, num_subcores=NS)
    sc_params = pltpu.CompilerParams(needs_layout_passes=False,
                                     use_tc_tiling_on_sc=False)

    pad = NNZ_PAD - NNZ
    colp = jnp.concatenate([col, jnp.zeros((pad,), jnp.int32)])
    rowp = jnp.concatenate([row, jnp.full((pad,), N_OUT, jnp.int32)])
    datap = jnp.concatenate([data, jnp.zeros((pad,), jnp.float32)])

    pool = pl.kernel(
        _pool_body,
        out_type=jax.ShapeDtypeStruct((B, NC, APAD, C), jnp.float32),
        mesh=mesh,
        compiler_params=sc_params,
        scratch_types=[
            pltpu.VMEM_SHARED((APAD, C), jnp.float32),
            pltpu.VMEM((ECH, C), jnp.float32),
            pltpu.VMEM((ECH,), jnp.int32),
            pltpu.VMEM((ECH,), jnp.int32),
            pltpu.VMEM((ECH,), jnp.float32),
            pltpu.VMEM((ECH,), jnp.int32),
            pltpu.SemaphoreType.DMA,
        ],
    )
    pooled = pool(x, colp, rowp, datap)
    pooled_flat = pooled.reshape(B, NC * APAD, C)

    idx_pad = jnp.zeros((VPAD, S10), jnp.int32).at[:N_OUT, :S].set(indices)
    idx_flat = idx_pad.reshape(VPAD, KS, 2).transpose(1, 0, 2).reshape(-1)

    spiral = pl.kernel(
        _spiral_body,
        out_type=jax.ShapeDtypeStruct((B, RP10, C), jnp.float32),
        mesh=mesh,
        compiler_params=sc_params,
        scratch_types=[
            pltpu.VMEM((RCH,), jnp.int32),
            pltpu.VMEM((RCH, C), jnp.float32),
            pltpu.SemaphoreType.DMA,
        ],
    )
    g = spiral(pooled_flat, idx_flat).reshape(B, KS, VPAD, 2 * C)

    w_pad = jnp.zeros((KS * 2 * C, C), jnp.float32).at[:S * C].set(W)
    w3 = w_pad.reshape(KS, 2 * C, C)

    out = pl.pallas_call(
        _dense_body,
        grid=(B, N_OUT // VB),
        in_specs=[
            pl.BlockSpec((1, KS, VB, 2 * C), lambda bb, i: (bb, 0, i, 0)),
            pl.BlockSpec((KS, 2 * C, C), lambda bb, i: (0, 0, 0)),
            pl.BlockSpec((1, C), lambda bb, i: (0, 0)),
        ],
        out_specs=pl.BlockSpec((1, VB, C), lambda bb, i: (bb, i, 0)),
        out_shape=jax.ShapeDtypeStruct((B, N_OUT, C), jnp.float32),
    )(g, w3, b.reshape(1, C))
    return out


# speedup vs baseline: 4.1142x; 1.1010x over previous
import functools

import jax
import jax.numpy as jnp
from jax import lax
from jax.experimental import pallas as pl
from jax.experimental.pallas import tpu as pltpu
from jax.experimental.pallas import tpu_sc as plsc

B, N_IN, N_OUT, C, S = 4, 25000, 50000, 64, 9
NNZ = 100000
NC, NS, L = 2, 16, 16

HALF = N_OUT // NC
APAD = 25088
AROWS_T = APAD // NS
NNZ_PAD = 102400
E_T = NNZ_PAD // NS
ECH = 256
S10 = 10
VPAD = 51200
RP10 = VPAD * S10
R_T = RP10 // (NC * NS)
RCH = 400
GROWS = B * VPAD * S10 // 2
VB = 400
KS = S10 // 2


def _pool_body(x_hbm, colp, rowp, datap, pooled_hbm,
               acc, xrows, colbuf, rowbuf, databuf, lrbuf, sem):
    c = lax.axis_index("c")
    t = lax.axis_index("s")
    base_local = HALF * c

    zv = jnp.zeros((L,), jnp.float32)
    for b in range(B):
        for r in range(ECH):
            for j in range(C // L):
                xrows[r, pl.ds(j * L, L)] = zv
        for z6 in range(AROWS_T // ECH):
            pltpu.sync_copy(xrows, acc.at[pl.ds(t * AROWS_T + z6 * ECH, ECH)])
        pltpu.sync_copy(
            xrows.at[pl.ds(0, AROWS_T % ECH)],
            acc.at[pl.ds(t * AROWS_T + (AROWS_T // ECH) * ECH, AROWS_T % ECH)])
        plsc.subcore_barrier()

        @pl.loop(0, E_T // ECH)
        def _chunk(k):
            ebase = t * E_T + k * ECH
            pltpu.sync_copy(colp.at[pl.ds(ebase, ECH)], colbuf)
            pltpu.sync_copy(rowp.at[pl.ds(ebase, ECH)], rowbuf)
            pltpu.sync_copy(datap.at[pl.ds(ebase, ECH)], databuf)
            pltpu.async_copy(x_hbm.at[b].at[colbuf], xrows, sem).wait()
            for i in range(ECH // L):
                rv = rowbuf[pl.ds(i * L, L)]
                lr = rv - base_local
                ok = (lr >= 0) & (lr < HALF)
                lrbuf[pl.ds(i * L, L)] = jnp.where(ok, lr, HALF)
            @pl.loop(0, ECH)
            def _edge(e):
                dvec = plsc.load_gather(databuf, [jnp.full((L,), e, jnp.int32)])
                for j in range(C // L):
                    xrows[e, pl.ds(j * L, L)] = xrows[e, pl.ds(j * L, L)] * dvec
            pltpu.sync_copy(xrows, acc.at[lrbuf], add=True)

        plsc.subcore_barrier()
        pltpu.sync_copy(acc.at[pl.ds(t * AROWS_T, AROWS_T)],
                        pooled_hbm.at[b].at[c].at[pl.ds(t * AROWS_T, AROWS_T)])


def _spiral_body(pooled_hbm, idx_hbm, g_hbm, idxbig, g0, g1, g2, g3, sem):
    c = lax.axis_index("c")
    t = lax.axis_index("s")
    wid = t * NC + c
    rbase = wid * R_T
    gbufs = (g0, g1, g2, g3)

    pltpu.sync_copy(idx_hbm.at[pl.ds(rbase, R_T)], idxbig)

    @pl.loop(0, R_T // L)
    def _remap(i):
        iv = idxbig[pl.ds(i * L, L)]
        idxbig[pl.ds(i * L, L)] = jnp.where(iv >= HALF, iv + (APAD - HALF), iv)

    @pl.loop(0, R_T // RCH)
    def _chunk(k):
        base = rbase + k * RCH
        idxv = idxbig.at[pl.ds(k * RCH, RCH)]
        cps = [pltpu.async_copy(pooled_hbm.at[b].at[idxv], gbufs[b], sem)
               for b in range(B)]
        for b in range(B):
            cps[b].wait()
        for b in range(B):
            pltpu.sync_copy(gbufs[b], g_hbm.at[b].at[pl.ds(base, RCH)])


def _dense_body(g_ref, w_ref, bias_ref, out_ref):
    z = jax.lax.dot_general(g_ref[0, 0], w_ref[0], (((1,), (0,)), ((), ())),
                            preferred_element_type=jnp.float32)
    for ks in range(1, KS):
        z = z + jax.lax.dot_general(g_ref[0, ks], w_ref[ks],
                                    (((1,), (0,)), ((), ())),
                                    preferred_element_type=jnp.float32)
    z = z + bias_ref[...]
    out_ref[0] = jnp.where(z > 0, z, jnp.exp(z) - 1.0)


@functools.partial(jax.jit, static_argnums=())
def kernel(x, row, col, data, indices, W, b):
    mesh = plsc.VectorSubcoreMesh(core_axis_name="c", subcore_axis_name="s",
                                  num_cores=NC, num_subcores=NS)
    sc_params = pltpu.CompilerParams(needs_layout_passes=False,
                                     use_tc_tiling_on_sc=False)

    pad = NNZ_PAD - NNZ
    colp = jnp.concatenate([col, jnp.zeros((pad,), jnp.int32)])
    rowp = jnp.concatenate([row, jnp.full((pad,), N_OUT, jnp.int32)])
    datap = jnp.concatenate([data, jnp.zeros((pad,), jnp.float32)])

    pool = pl.kernel(
        _pool_body,
        out_type=jax.ShapeDtypeStruct((B, NC, APAD, C), jnp.float32),
        mesh=mesh,
        compiler_params=sc_params,
        scratch_types=[
            pltpu.VMEM_SHARED((APAD, C), jnp.float32),
            pltpu.VMEM((ECH, C), jnp.float32),
            pltpu.VMEM((ECH,), jnp.int32),
            pltpu.VMEM((ECH,), jnp.int32),
            pltpu.VMEM((ECH,), jnp.float32),
            pltpu.VMEM((ECH,), jnp.int32),
            pltpu.SemaphoreType.DMA,
        ],
    )
    pooled = pool(x, colp, rowp, datap)
    pooled_flat = pooled.reshape(B, NC * APAD, C)

    idx_pad = jnp.zeros((VPAD, S10), jnp.int32).at[:N_OUT, :S].set(indices)
    idx_flat = idx_pad.reshape(VPAD, KS, 2).transpose(1, 0, 2).reshape(-1)

    spiral = pl.kernel(
        _spiral_body,
        out_type=jax.ShapeDtypeStruct((B, RP10, C), jnp.float32),
        mesh=mesh,
        compiler_params=sc_params,
        scratch_types=[
            pltpu.VMEM((R_T,), jnp.int32),
            pltpu.VMEM((RCH, C), jnp.float32),
            pltpu.VMEM((RCH, C), jnp.float32),
            pltpu.VMEM((RCH, C), jnp.float32),
            pltpu.VMEM((RCH, C), jnp.float32),
            pltpu.SemaphoreType.DMA,
        ],
    )
    g = spiral(pooled_flat, idx_flat).reshape(B, KS, VPAD, 2 * C)

    w_pad = jnp.zeros((KS * 2 * C, C), jnp.float32).at[:S * C].set(W)
    w3 = w_pad.reshape(KS, 2 * C, C)

    out = pl.pallas_call(
        _dense_body,
        grid=(B, N_OUT // VB),
        in_specs=[
            pl.BlockSpec((1, KS, VB, 2 * C), lambda bb, i: (bb, 0, i, 0)),
            pl.BlockSpec((KS, 2 * C, C), lambda bb, i: (0, 0, 0)),
            pl.BlockSpec((1, C), lambda bb, i: (0, 0)),
        ],
        out_specs=pl.BlockSpec((1, VB, C), lambda bb, i: (bb, i, 0)),
        out_shape=jax.ShapeDtypeStruct((B, N_OUT, C), jnp.float32),
    )(g, w3, b.reshape(1, C))
    return out


# speedup vs baseline: 6.9430x; 1.6875x over previous
import functools

import jax
import jax.numpy as jnp
from jax import lax
from jax.experimental import pallas as pl
from jax.experimental.pallas import tpu as pltpu
from jax.experimental.pallas import tpu_sc as plsc

B, N_IN, N_OUT, C, S = 4, 25000, 50000, 64, 9
NNZ = 100000
NC, NS, L = 2, 16, 16
BP = B // 2
C2 = 2 * C

HALF = N_OUT // NC
APAD = 25088
AROWS_T = APAD // NS
M = NC * APAD
NNZ_PAD = 102400
E_T = NNZ_PAD // NS
ECH = 128
VPAD = 51200
RP9 = VPAD * S
R_T = RP9 // (NC * NS)
RCH = 144
VB = 400


def _pool_body(xq_hbm, epk_hbm, pooled_hbm,
               acc, xr0, xr1, ep0, ep1, cb0, cb1, db0, db1, lr0, lr1, sem):
    c = lax.axis_index("c")
    t = lax.axis_index("s")
    base_local = HALF * c
    xrs, eps, cbs, dbs, lrs = (xr0, xr1), (ep0, ep1), (cb0, cb1), (db0, db1), (lr0, lr1)

    zv = jnp.zeros((2 * L,), jnp.bfloat16)
    for bp in range(BP):
        for r in range(ECH):
            for j in range(C2 // (2 * L)):
                xr0[r, pl.ds(j * 2 * L, 2 * L)] = zv
        for z6 in range(AROWS_T // ECH):
            pltpu.sync_copy(xr0, acc.at[pl.ds(t * AROWS_T + z6 * ECH, ECH)])
        pltpu.sync_copy(
            xr0.at[pl.ds(0, AROWS_T % ECH)],
            acc.at[pl.ds(t * AROWS_T + (AROWS_T // ECH) * ECH, AROWS_T % ECH)])
        plsc.subcore_barrier()

        @pl.loop(0, E_T // ECH // 2)
        def _pairs(kk):
            cps = []
            for p in range(2):
                ebase = t * E_T + (kk * 2 + p) * ECH
                pltpu.sync_copy(epk_hbm.at[pl.ds(ebase * 4, ECH * 4)], eps[p])
                for i in range(ECH // L):
                    evec4 = (lax.iota(jnp.int32, L) + i * L) * 4
                    colv = plsc.load_gather(eps[p], [evec4])
                    rowv = plsc.load_gather(eps[p], [evec4 + 1])
                    dv = plsc.load_gather(eps[p], [evec4 + 2])
                    cbs[p][pl.ds(i * L, L)] = colv
                    lr = rowv - base_local
                    ok = (lr >= 0) & (lr < HALF)
                    lrs[p][pl.ds(i * L, L)] = jnp.where(ok, lr, HALF)
                    dbs[p][pl.ds(i * L, L)] = plsc.bitcast(dv, jnp.float32)
                cps.append(pltpu.async_copy(
                    xq_hbm.at[bp].at[cbs[p]], xrs[p], sem))
            for p in range(2):
                cps[p].wait()

                @pl.loop(0, ECH)
                def _edge(e):
                    d16 = plsc.load_gather(dbs[p], [jnp.full((L,), e, jnp.int32)])
                    dvec = plsc.pack(d16, d16, format=plsc.PackFormat.INTERLEAVED)
                    for j in range(C2 // (2 * L)):
                        sl = pl.ds(j * 2 * L, 2 * L)
                        xrs[p][e, sl] = xrs[p][e, sl] * dvec
                pltpu.sync_copy(xrs[p], acc.at[lrs[p]], add=True)

        plsc.subcore_barrier()
        pltpu.sync_copy(acc.at[pl.ds(t * AROWS_T, AROWS_T)],
                        pooled_hbm.at[bp].at[pl.ds(c * APAD + t * AROWS_T,
                                                   AROWS_T)])


def _spiral_body(pooled_hbm, idx_hbm, g_hbm,
                 ic0, ic1, i20, i21, gf0, gf1, gu0, gu1, sem):
    c = lax.axis_index("c")
    t = lax.axis_index("s")
    wid = t * NC + c
    rbase = wid * R_T
    ics, i2s, gfs, gus = (ic0, ic1), (i20, i21), (gf0, gf1), (gu0, gu1)

    @pl.loop(0, R_T // RCH // 2)
    def _pairs(kk):
        cps = []
        for p in range(2):
            k = kk * 2 + p
            pltpu.sync_copy(idx_hbm.at[pl.ds(rbase + k * RCH, RCH)], ics[p])
            for i in range(RCH // L):
                evec = lax.iota(jnp.int32, L) + i * L
                iv = ics[p][pl.ds(i * L, L)]
                iv = jnp.where(iv >= HALF, iv + (APAD - HALF), iv)
                plsc.store_scatter(i2s[p], [evec * 2], iv)
                plsc.store_scatter(i2s[p], [evec * 2 + 1], iv + M)
            cps.append(pltpu.async_copy(pooled_hbm.at[i2s[p]], gfs[p], sem))
        for p in range(2):
            cps[p].wait()

            @pl.loop(0, 2 * RCH)
            def _row(r):
                for j in range(C // L):
                    w16 = gfs[p][r, pl.ds(j * L, L)]
                    b32 = plsc.bitcast(w16, jnp.bfloat16)
                    lo, hi = plsc.unpack(b32, format=plsc.PackFormat.INTERLEAVED)
                    pos = r * C2 + j * 2 * L + lax.iota(jnp.int32, L) * 2
                    plsc.store_scatter(gus[p], [pos], lo)
                    plsc.store_scatter(gus[p], [pos + 1], hi)
            pltpu.sync_copy(
                gus[p],
                g_hbm.at[pl.ds((rbase + (kk * 2 + p) * RCH) * 2 * C2,
                               2 * RCH * C2)])


def _dense_body(*refs):
    g_refs, w_ref, bias_ref, out_ref = refs[:S], refs[S], refs[S + 1], refs[S + 2]
    z = jax.lax.dot_general(g_refs[0][...], w_ref[0], (((1,), (0,)), ((), ())),
                            preferred_element_type=jnp.float32)
    for s in range(1, S):
        z = z + jax.lax.dot_general(g_refs[s][...], w_ref[s],
                                    (((1,), (0,)), ((), ())),
                                    preferred_element_type=jnp.float32)
    z = z + bias_ref[...]
    out_ref[...] = jnp.where(z > 0, z, jnp.exp(z) - 1.0)


@functools.partial(jax.jit, static_argnums=())
def kernel(x, row, col, data, indices, W, b):
    mesh = plsc.VectorSubcoreMesh(core_axis_name="c", subcore_axis_name="s",
                                  num_cores=NC, num_subcores=NS)
    sc_params = pltpu.CompilerParams(needs_layout_passes=False,
                                     use_tc_tiling_on_sc=False)

    xq = x.astype(jnp.bfloat16).reshape(BP, 2, N_IN, C).transpose(0, 2, 1, 3)
    xq = xq.reshape(BP, N_IN, C2)

    pad = NNZ_PAD - NNZ
    colp = jnp.concatenate([col, jnp.zeros((pad,), jnp.int32)])
    rowp = jnp.concatenate([row, jnp.full((pad,), N_OUT, jnp.int32)])
    datap = jnp.concatenate([data, jnp.zeros((pad,), jnp.float32)])
    dbits = jax.lax.bitcast_convert_type(datap, jnp.int32)
    epk = jnp.stack([colp, rowp, dbits, jnp.zeros_like(colp)], axis=1)
    epk = epk.reshape(-1)

    pool = pl.kernel(
        _pool_body,
        out_type=jax.ShapeDtypeStruct((BP, M, C2), jnp.bfloat16),
        mesh=mesh,
        compiler_params=sc_params,
        scratch_types=[
            pltpu.VMEM_SHARED((APAD, C2), jnp.bfloat16),
            pltpu.VMEM((ECH, C2), jnp.bfloat16),
            pltpu.VMEM((ECH, C2), jnp.bfloat16),
            pltpu.VMEM((ECH * 4,), jnp.int32),
            pltpu.VMEM((ECH * 4,), jnp.int32),
            pltpu.VMEM((ECH,), jnp.int32),
            pltpu.VMEM((ECH,), jnp.int32),
            pltpu.VMEM((ECH,), jnp.float32),
            pltpu.VMEM((ECH,), jnp.float32),
            pltpu.VMEM((ECH,), jnp.int32),
            pltpu.VMEM((ECH,), jnp.int32),
            pltpu.SemaphoreType.DMA,
        ],
    )
    pooled2 = pool(xq, epk)

    p2f = jax.lax.bitcast_convert_type(
        pooled2.reshape(BP, M, C, 2), jnp.float32).reshape(BP * M, C)

    idx_pad = jnp.zeros((VPAD, S), jnp.int32).at[:N_OUT].set(indices)
    idx_flat = idx_pad.T.reshape(-1)

    spiral = pl.kernel(
        _spiral_body,
        out_type=jax.ShapeDtypeStruct((2 * RP9 * C2,), jnp.float32),
        mesh=mesh,
        compiler_params=sc_params,
        scratch_types=[
            pltpu.VMEM((RCH,), jnp.int32),
            pltpu.VMEM((RCH,), jnp.int32),
            pltpu.VMEM((2 * RCH,), jnp.int32),
            pltpu.VMEM((2 * RCH,), jnp.int32),
            pltpu.VMEM((2 * RCH, C), jnp.float32),
            pltpu.VMEM((2 * RCH, C), jnp.float32),
            pltpu.VMEM((2 * RCH * C2,), jnp.float32),
            pltpu.VMEM((2 * RCH * C2,), jnp.float32),
            pltpu.SemaphoreType.DMA,
        ],
    )
    g = spiral(p2f, idx_flat).reshape(2 * RP9, C2)

    ws = W.reshape(S, C, C)
    w2 = jnp.zeros((S, 2, C, 2, C), jnp.float32)
    w2 = w2.at[:, 0, :, 0, :].set(ws).at[:, 1, :, 1, :].set(ws)
    w2 = w2.reshape(S, C2, C2)
    bias2 = jnp.concatenate([b, b]).reshape(1, C2)

    nvb = VPAD // VB
    in_specs = [pl.BlockSpec((2 * VB, C2), (lambda i, s=s: (s * nvb + i, 0)))
                for s in range(S)]
    in_specs.append(pl.BlockSpec((S, C2, C2), lambda i: (0, 0, 0)))
    in_specs.append(pl.BlockSpec((1, C2), lambda i: (0, 0)))

    out2 = pl.pallas_call(
        _dense_body,
        grid=(N_OUT // VB,),
        in_specs=in_specs,
        out_specs=pl.BlockSpec((2 * VB, C2), lambda i: (i, 0)),
        out_shape=jax.ShapeDtypeStruct((2 * N_OUT, C2), jnp.float32),
    )(*([g] * S), w2, bias2)

    out = out2.reshape(N_OUT, 2, 2, C).transpose(1, 2, 0, 3)
    return out.reshape(B, N_OUT, C)


# speedup vs baseline: 7.2420x; 1.0431x over previous
import functools

import jax
import jax.numpy as jnp
from jax import lax
from jax.experimental import pallas as pl
from jax.experimental.pallas import tpu as pltpu
from jax.experimental.pallas import tpu_sc as plsc

B, N_IN, N_OUT, C, S = 4, 25000, 50000, 64, 9
NNZ = 100000
NC, NS, L = 2, 16, 16
BP = B // 2
C2 = 2 * C

HALF = N_OUT // NC
APAD = 25088
AROWS_T = APAD // NS
M = NC * APAD
NNZ_PAD = 102400
E_T = NNZ_PAD // NS
ECH = 128
VPAD = 51200
RP9 = VPAD * S
R_T = RP9 // (NC * NS)
RCH = 144
VB = 400


def _pool_body(xq_hbm, epk_hbm, pooled_hbm,
               acc, xr0, xr1, ep0, ep1, cb0, cb1, db0, db1, lr0, lr1, sem):
    c = lax.axis_index("c")
    t = lax.axis_index("s")
    base_local = HALF * c
    xrs, eps, cbs, dbs, lrs = (xr0, xr1), (ep0, ep1), (cb0, cb1), (db0, db1), (lr0, lr1)

    zv = jnp.zeros((2 * L,), jnp.bfloat16)
    for bp in range(BP):
        for r in range(ECH):
            for j in range(C2 // (2 * L)):
                xr0[r, pl.ds(j * 2 * L, 2 * L)] = zv
        for z6 in range(AROWS_T // ECH):
            pltpu.sync_copy(xr0, acc.at[pl.ds(t * AROWS_T + z6 * ECH, ECH)])
        pltpu.sync_copy(
            xr0.at[pl.ds(0, AROWS_T % ECH)],
            acc.at[pl.ds(t * AROWS_T + (AROWS_T // ECH) * ECH, AROWS_T % ECH)])
        plsc.subcore_barrier()

        @pl.loop(0, E_T // ECH // 2)
        def _pairs(kk):
            cps = []
            for p in range(2):
                ebase = t * E_T + (kk * 2 + p) * ECH
                pltpu.sync_copy(epk_hbm.at[pl.ds(ebase * 4, ECH * 4)], eps[p])
                for i in range(ECH // L):
                    evec4 = (lax.iota(jnp.int32, L) + i * L) * 4
                    colv = plsc.load_gather(eps[p], [evec4])
                    rowv = plsc.load_gather(eps[p], [evec4 + 1])
                    dv = plsc.load_gather(eps[p], [evec4 + 2])
                    cbs[p][pl.ds(i * L, L)] = colv
                    lr = rowv - base_local
                    ok = (lr >= 0) & (lr < HALF)
                    lrs[p][pl.ds(i * L, L)] = jnp.where(ok, lr, HALF)
                    dbs[p][pl.ds(i * L, L)] = plsc.bitcast(dv, jnp.float32)
                cps.append(pltpu.async_copy(
                    xq_hbm.at[bp].at[cbs[p]], xrs[p], sem))
            for p in range(2):
                cps[p].wait()

                @pl.loop(0, ECH)
                def _edge(e):
                    d16 = plsc.load_gather(dbs[p], [jnp.full((L,), e, jnp.int32)])
                    dvec = plsc.pack(d16, d16, format=plsc.PackFormat.INTERLEAVED)
                    for j in range(C2 // (2 * L)):
                        sl = pl.ds(j * 2 * L, 2 * L)
                        xrs[p][e, sl] = xrs[p][e, sl] * dvec
                pltpu.sync_copy(xrs[p], acc.at[lrs[p]], add=True)

        plsc.subcore_barrier()
        pltpu.sync_copy(acc.at[pl.ds(t * AROWS_T, AROWS_T)],
                        pooled_hbm.at[bp].at[pl.ds(c * APAD + t * AROWS_T,
                                                   AROWS_T)])


def _spiral_body(pooled_hbm, idx_hbm, g_hbm,
                 ic0, ic1, i20, i21, gf0, gf1, gu0, gu1, sem0, sem1):
    c = lax.axis_index("c")
    t = lax.axis_index("s")
    wid = t * NC + c
    rbase = wid * R_T
    ics, i2s, gfs, gus = (ic0, ic1), (i20, i21), (gf0, gf1), (gu0, gu1)
    sems = (sem0, sem1)
    nch = R_T // RCH

    def fire(p, k):
        pltpu.sync_copy(idx_hbm.at[pl.ds(rbase + k * RCH, RCH)], ics[p])
        for i in range(RCH // L):
            evec = lax.iota(jnp.int32, L) + i * L
            iv = ics[p][pl.ds(i * L, L)]
            iv = jnp.where(iv >= HALF, iv + (APAD - HALF), iv)
            plsc.store_scatter(i2s[p], [evec * 2], iv)
            plsc.store_scatter(i2s[p], [evec * 2 + 1], iv + M)
        pltpu.async_copy(pooled_hbm.at[i2s[p]], gfs[p], sems[p])

    def drain(p, k):
        pltpu.make_async_copy(pooled_hbm.at[i2s[p]], gfs[p], sems[p]).wait()

        @pl.loop(0, 2 * RCH)
        def _row(r):
            for j in range(C // L):
                w16 = gfs[p][r, pl.ds(j * L, L)]
                b32 = plsc.bitcast(w16, jnp.bfloat16)
                lo, hi = plsc.unpack(b32, format=plsc.PackFormat.INTERLEAVED)
                pos = r * C2 + j * 2 * L + lax.iota(jnp.int32, L) * 2
                plsc.store_scatter(gus[p], [pos], lo)
                plsc.store_scatter(gus[p], [pos + 1], hi)
        pltpu.sync_copy(gus[p],
                        g_hbm.at[pl.ds((rbase + k * RCH) * 2 * C2,
                                       2 * RCH * C2)])

    for p in range(2):
        fire(p, p)

    @pl.loop(0, nch // 2 - 1)
    def _steady(kk):
        for p in range(2):
            drain(p, kk * 2 + p)
            fire(p, kk * 2 + 2 + p)

    for p in range(2):
        drain(p, nch - 2 + p)


def _dense_body(*refs):
    g_refs, w_ref, bias_ref, out_ref = refs[:S], refs[S], refs[S + 1], refs[S + 2]
    z = jax.lax.dot_general(g_refs[0][...], w_ref[0], (((1,), (0,)), ((), ())),
                            preferred_element_type=jnp.float32)
    for s in range(1, S):
        z = z + jax.lax.dot_general(g_refs[s][...], w_ref[s],
                                    (((1,), (0,)), ((), ())),
                                    preferred_element_type=jnp.float32)
    z = z + bias_ref[...]
    out_ref[...] = jnp.where(z > 0, z, jnp.exp(z) - 1.0)


@functools.partial(jax.jit, static_argnums=())
def kernel(x, row, col, data, indices, W, b):
    mesh = plsc.VectorSubcoreMesh(core_axis_name="c", subcore_axis_name="s",
                                  num_cores=NC, num_subcores=NS)
    sc_params = pltpu.CompilerParams(needs_layout_passes=False,
                                     use_tc_tiling_on_sc=False)

    xq = x.astype(jnp.bfloat16).reshape(BP, 2, N_IN, C).transpose(0, 2, 1, 3)
    xq = xq.reshape(BP, N_IN, C2)

    pad = NNZ_PAD - NNZ
    colp = jnp.concatenate([col, jnp.zeros((pad,), jnp.int32)])
    rowp = jnp.concatenate([row, jnp.full((pad,), N_OUT, jnp.int32)])
    datap = jnp.concatenate([data, jnp.zeros((pad,), jnp.float32)])
    dbits = jax.lax.bitcast_convert_type(datap, jnp.int32)
    epk = jnp.stack([colp, rowp, dbits, jnp.zeros_like(colp)], axis=1)
    epk = epk.reshape(-1)

    pool = pl.kernel(
        _pool_body,
        out_type=jax.ShapeDtypeStruct((BP, M, C2), jnp.bfloat16),
        mesh=mesh,
        compiler_params=sc_params,
        scratch_types=[
            pltpu.VMEM_SHARED((APAD, C2), jnp.bfloat16),
            pltpu.VMEM((ECH, C2), jnp.bfloat16),
            pltpu.VMEM((ECH, C2), jnp.bfloat16),
            pltpu.VMEM((ECH * 4,), jnp.int32),
            pltpu.VMEM((ECH * 4,), jnp.int32),
            pltpu.VMEM((ECH,), jnp.int32),
            pltpu.VMEM((ECH,), jnp.int32),
            pltpu.VMEM((ECH,), jnp.float32),
            pltpu.VMEM((ECH,), jnp.float32),
            pltpu.VMEM((ECH,), jnp.int32),
            pltpu.VMEM((ECH,), jnp.int32),
            pltpu.SemaphoreType.DMA,
        ],
    )
    pooled2 = pool(xq, epk)

    p2f = jax.lax.bitcast_convert_type(
        pooled2.reshape(BP, M, C, 2), jnp.float32).reshape(BP * M, C)

    idx_pad = jnp.zeros((VPAD, S), jnp.int32).at[:N_OUT].set(indices)
    idx_flat = idx_pad.T.reshape(-1)

    spiral = pl.kernel(
        _spiral_body,
        out_type=jax.ShapeDtypeStruct((2 * RP9 * C2,), jnp.float32),
        mesh=mesh,
        compiler_params=sc_params,
        scratch_types=[
            pltpu.VMEM((RCH,), jnp.int32),
            pltpu.VMEM((RCH,), jnp.int32),
            pltpu.VMEM((2 * RCH,), jnp.int32),
            pltpu.VMEM((2 * RCH,), jnp.int32),
            pltpu.VMEM((2 * RCH, C), jnp.float32),
            pltpu.VMEM((2 * RCH, C), jnp.float32),
            pltpu.VMEM((2 * RCH * C2,), jnp.float32),
            pltpu.VMEM((2 * RCH * C2,), jnp.float32),
            pltpu.SemaphoreType.DMA,
            pltpu.SemaphoreType.DMA,
        ],
    )
    g = spiral(p2f, idx_flat).reshape(2 * RP9, C2)

    ws = W.reshape(S, C, C)
    w2 = jnp.zeros((S, 2, C, 2, C), jnp.float32)
    w2 = w2.at[:, 0, :, 0, :].set(ws).at[:, 1, :, 1, :].set(ws)
    w2 = w2.reshape(S, C2, C2)
    bias2 = jnp.concatenate([b, b]).reshape(1, C2)

    nvb = VPAD // VB
    in_specs = [pl.BlockSpec((2 * VB, C2), (lambda i, s=s: (s * nvb + i, 0)))
                for s in range(S)]
    in_specs.append(pl.BlockSpec((S, C2, C2), lambda i: (0, 0, 0)))
    in_specs.append(pl.BlockSpec((1, C2), lambda i: (0, 0)))

    out2 = pl.pallas_call(
        _dense_body,
        grid=(N_OUT // VB,),
        in_specs=in_specs,
        out_specs=pl.BlockSpec((2 * VB, C2), lambda i: (i, 0)),
        out_shape=jax.ShapeDtypeStruct((2 * N_OUT, C2), jnp.float32),
    )(*([g] * S), w2, bias2)

    out = out2.reshape(N_OUT, 2, 2, C).transpose(1, 2, 0, 3)
    return out.reshape(B, N_OUT, C)
